# Initial kernel scaffold; baseline (speedup 1.0000x reference)
import jax
import jax.numpy as jnp
from jax.experimental import pallas as pl


def kernel(feature_vectors, embeddings):
    raise NotImplementedError("write your pallas kernel here")


# speedup vs baseline: 1.6787x; 1.6787x over previous
import jax
import jax.numpy as jnp
from jax.experimental import pallas as pl

_HI = jax.lax.Precision.HIGHEST


def _exact_d2(x, g):
    diff = x - g
    t = diff * diff
    M = t.shape[0]
    t3 = t.reshape(M, 8, 8)
    p = t3[:, 0, :]
    for gi in range(1, 8):
        p = p + t3[:, gi, :]
    a0 = p[:, 0:1] + p[:, 4:5]
    a2 = p[:, 2:3] + p[:, 6:7]
    a1 = p[:, 1:2] + p[:, 5:6]
    a3 = p[:, 3:4] + p[:, 7:8]
    return (a0 + a2) + (a1 + a3)


def _vq_kernel(x_ref, e_ref, et_ref, en2_ref, out_ref, idx_ref):
    x = x_ref[...]
    e = e_ref[...]
    et = et_ref[...]
    en2 = en2_ref[...]
    scores = en2 - 2.0 * jnp.dot(
        x, e, preferred_element_type=jnp.float32, precision=_HI)
    M, K = scores.shape
    kiota = jax.lax.broadcasted_iota(jnp.int32, (M, K), 1)
    i1 = jnp.argmin(scores, axis=-1).astype(jnp.int32)
    masked = jnp.where(kiota == i1[:, None], jnp.inf, scores)
    i2 = jnp.argmin(masked, axis=-1).astype(jnp.int32)

    oh1 = (kiota == i1[:, None]).astype(jnp.float32)
    oh2 = (kiota == i2[:, None]).astype(jnp.float32)
    g1 = jnp.dot(oh1, et, preferred_element_type=jnp.float32, precision=_HI)
    g2 = jnp.dot(oh2, et, preferred_element_type=jnp.float32, precision=_HI)

    d1 = jnp.sqrt(_exact_d2(x, g1))
    d2 = jnp.sqrt(_exact_d2(x, g2))
    i1c = i1[:, None]
    i2c = i2[:, None]
    swap = (d2 < d1) | ((d2 == d1) & (i2c < i1c))
    idx_ref[...] = jnp.where(swap, i2c, i1c)
    out_ref[...] = jnp.where(swap, g2, g1)


def kernel(feature_vectors, embeddings):
    B, D, N = feature_vectors.shape
    K = embeddings.shape[1]
    M = B * N
    x = feature_vectors.transpose(0, 2, 1).reshape(M, D)
    et = embeddings.T
    en2 = jnp.sum(embeddings * embeddings, axis=0, keepdims=True)

    out, idx = pl.pallas_call(
        _vq_kernel,
        out_shape=(
            jax.ShapeDtypeStruct((M, D), jnp.float32),
            jax.ShapeDtypeStruct((M, 1), jnp.int32),
        ),
    )(x, embeddings, et, en2)

    closest = out.reshape(B, N, D).transpose(0, 2, 1)
    closest_indices = idx.reshape(B, N)
    return closest, closest_indices


# speedup vs baseline: 3.8392x; 2.2870x over previous
import jax
import jax.numpy as jnp
from jax.experimental import pallas as pl

_HI = jax.lax.Precision.HIGHEST


def _exact_d2(xb, g):
    diff = xb - g
    t = diff * diff
    p = t[0:8, :]
    for gi in range(1, 8):
        p = p + t[8 * gi:8 * gi + 8, :]
    a0 = p[0:1, :] + p[4:5, :]
    a2 = p[2:3, :] + p[6:7, :]
    a1 = p[1:2, :] + p[5:6, :]
    a3 = p[3:4, :] + p[7:8, :]
    return (a0 + a2) + (a1 + a3)


def _vq_kernel(x_ref, e_ref, out_ref, idx_ref):
    e = e_ref[...]
    en2 = jnp.sum(e * e, axis=0, keepdims=True)
    B = x_ref.shape[0]
    N = x_ref.shape[2]
    K = e.shape[1]
    for b in range(B):
        xb = x_ref[b]
        prod = jax.lax.dot_general(
            xb, e, (((0,), (0,)), ((), ())),
            preferred_element_type=jnp.float32, precision=_HI)
        scores = en2 - 2.0 * prod
        kiota = jax.lax.broadcasted_iota(jnp.int32, (N, K), 1)
        i1 = jnp.argmin(scores, axis=-1).astype(jnp.int32)
        masked = jnp.where(kiota == i1[:, None], jnp.inf, scores)
        i2 = jnp.argmin(masked, axis=-1).astype(jnp.int32)

        niota = jax.lax.broadcasted_iota(jnp.int32, (K, N), 0)
        oh1 = (niota == i1[None, :]).astype(jnp.float32)
        oh2 = (niota == i2[None, :]).astype(jnp.float32)
        g1 = jnp.dot(e, oh1, preferred_element_type=jnp.float32,
                     precision=_HI)
        g2 = jnp.dot(e, oh2, preferred_element_type=jnp.float32,
                     precision=_HI)

        d1 = jnp.sqrt(_exact_d2(xb, g1))
        d2 = jnp.sqrt(_exact_d2(xb, g2))
        i1r = i1[None, :]
        i2r = i2[None, :]
        swap = (d2 < d1) | ((d2 == d1) & (i2r < i1r))
        idx_ref[b] = jnp.where(swap, i2r, i1r)[0]
        out_ref[b] = jnp.where(swap, g2, g1)


def kernel(feature_vectors, embeddings):
    B, D, N = feature_vectors.shape
    out, idx = pl.pallas_call(
        _vq_kernel,
        out_shape=(
            jax.ShapeDtypeStruct((B, D, N), jnp.float32),
            jax.ShapeDtypeStruct((B, N), jnp.int32),
        ),
    )(feature_vectors, embeddings)
    return out, idx
